# baseline (speedup 1.0000x reference)
import dataclasses
import functools

import jax
import jax.numpy as jnp
from jax import lax
from jax.experimental import pallas as pl
from jax.experimental.pallas import tpu as pltpu
from jax.experimental.pallas import tpu_sc as plsc

B = 16384
H = 128
NC = 2
NS = 16
NW = NC * NS
BPW = B // NW
CH = 128
NCH = BPW // CH
L = 16

P = 100000
PROWS = (P + H - 1) // H
FW = 2 * H

MAX_SLIP = 0.4
MAX_GUESS = 0.4
INV_T = 1.0 / 50.0


def _sc_gather(theta_table, fused_table, student_id, problem_id):
    mesh = plsc.VectorSubcoreMesh(core_axis_name="c", subcore_axis_name="s")
    cp = pltpu.CompilerParams()
    if "needs_layout_passes" in pltpu.CompilerParams.__dataclass_fields__:
        cp = dataclasses.replace(cp, needs_layout_passes=False)

    @functools.partial(
        pl.kernel,
        mesh=mesh,
        compiler_params=cp,
        out_type=[
            jax.ShapeDtypeStruct((B, H), jnp.float32),
            jax.ShapeDtypeStruct((B,), jnp.float32),
            jax.ShapeDtypeStruct((B,), jnp.float32),
        ],
        scratch_types=[
            pltpu.VMEM((BPW,), jnp.int32),
            pltpu.VMEM((BPW,), jnp.int32),
            pltpu.VMEM((BPW,), jnp.int32),
            pltpu.VMEM((CH, H), jnp.float32),
            pltpu.VMEM((CH, H), jnp.float32),
            pltpu.VMEM((CH, H), jnp.float32),
            pltpu.VMEM((CH, H), jnp.float32),
            pltpu.VMEM((CH, H), jnp.int32),
            pltpu.VMEM((CH, H), jnp.int32),
            pltpu.VMEM((BPW,), jnp.float32),
            pltpu.VMEM((BPW,), jnp.float32),
            pltpu.SemaphoreType.DMA,
            pltpu.SemaphoreType.DMA,
            pltpu.SemaphoreType.DMA,
            pltpu.SemaphoreType.DMA,
            pltpu.SemaphoreType.DMA,
            pltpu.SemaphoreType.DMA,
            pltpu.SemaphoreType.DMA,
        ],
    )
    def k(theta_hbm, fused_hbm, sid_hbm, pid_hbm,
          theta_out, slip_out, guess_out,
          sidx_v, pidx_v, ridx_v, tbuf0, tbuf1, tbuf2, tbuf3, fbuf0, fbuf1,
          slip_v, guess_v, tsem0, tsem1, tsem2, tsem3, fsem0, fsem1, wsem):
        wid = lax.axis_index("s") * NC + lax.axis_index("c")
        base = wid * BPW
        pltpu.sync_copy(sid_hbm.at[pl.ds(base, BPW)], sidx_v)
        pltpu.sync_copy(pid_hbm.at[pl.ds(base, BPW)], pidx_v)

        for g in range(BPW // L):
            pv = pidx_v[pl.ds(g * L, L)]
            ridx_v[pl.ds(g * L, L)] = lax.shift_right_logical(pv, 7)

        tbufs = (tbuf0, tbuf1, tbuf2, tbuf3)
        fbufs = (fbuf0, fbuf1)
        tsems = (tsem0, tsem1, tsem2, tsem3)
        fsems = (fsem0, fsem1)
        ht = [pltpu.async_copy(
                  theta_hbm.at[sidx_v.at[pl.ds(c * CH, CH)]],
                  tbufs[c], tsems[c])
              for c in range(NCH)]
        hf = [None] * NCH
        hf[0] = pltpu.async_copy(
            fused_hbm.at[ridx_v.at[pl.ds(0, CH)]], fbufs[0], fsems[0])
        wh = []
        for c in range(NCH):
            if c + 1 < NCH:
                hf[c + 1] = pltpu.async_copy(
                    fused_hbm.at[ridx_v.at[pl.ds((c + 1) * CH, CH)]],
                    fbufs[(c + 1) % 2], fsems[(c + 1) % 2])
            ht[c].wait()
            wh.append(pltpu.async_copy(
                tbufs[c], theta_out.at[pl.ds(base + c * CH, CH)], wsem))
            hf[c].wait()
            fb = fbufs[c % 2]
            for g in range(CH // L):
                pv = pidx_v[pl.ds(c * CH + g * L, L)]
                col = jnp.bitwise_and(pv, H - 1)
                rows = lax.iota(jnp.int32, L) + (g * L)
                v = plsc.load_gather(fb, [rows, col])
                slip_v[pl.ds(c * CH + g * L, L)] = plsc.bitcast(
                    lax.shift_left(v, 16), jnp.float32)
                guess_v[pl.ds(c * CH + g * L, L)] = plsc.bitcast(
                    jnp.bitwise_and(v, jnp.int32(-65536)), jnp.float32)
        pltpu.sync_copy(slip_v, slip_out.at[pl.ds(base, BPW)])
        pltpu.sync_copy(guess_v, guess_out.at[pl.ds(base, BPW)])
        for h in wh:
            h.wait()

    return k(theta_table, fused_table, student_id, problem_id)


_TC_BLK = 4096


def _tc_body(t_ref, k_ref, s_ref, g_ref, o_ref):
    th = t_ref[...]
    ke = k_ref[...]
    prod = ke * (jax.nn.sigmoid(th) - 0.5)
    ones = jnp.ones((H, 1), jnp.float32)
    n = lax.dot_general(prod, ones, (((1,), (0,)), ((), ())),
                        preferred_element_type=jnp.float32)
    n2 = n.reshape(_TC_BLK // H, H)
    p = jax.nn.sigmoid(n2 * INV_T)
    s = jax.nn.sigmoid(s_ref[...].reshape(_TC_BLK // H, H) * MAX_SLIP)
    g = jax.nn.sigmoid(g_ref[...].reshape(_TC_BLK // H, H) * MAX_GUESS)
    out = (1.0 - s) * p + g * (1.0 - p)
    o_ref[...] = out.reshape(_TC_BLK)


def _tc_math(theta_g, knowledge_emb, slip_g, guess_g):
    return pl.pallas_call(
        _tc_body,
        grid=(B // _TC_BLK,),
        in_specs=[
            pl.BlockSpec((_TC_BLK, H), lambda i: (i, 0)),
            pl.BlockSpec((_TC_BLK, H), lambda i: (i, 0)),
            pl.BlockSpec((_TC_BLK,), lambda i: (i,)),
            pl.BlockSpec((_TC_BLK,), lambda i: (i,)),
        ],
        out_specs=pl.BlockSpec((_TC_BLK,), lambda i: (i,)),
        out_shape=jax.ShapeDtypeStruct((B,), jnp.float32),
    )(theta_g, knowledge_emb, slip_g, guess_g)


def _build_fused(slip_table, guess_table):
    npad = PROWS * H - P
    su = lax.bitcast_convert_type(
        jnp.pad(slip_table[:, 0], (0, npad)), jnp.uint32)
    gu = lax.bitcast_convert_type(
        jnp.pad(guess_table[:, 0], (0, npad)), jnp.uint32)
    s16 = (su + jnp.uint32(0x8000)) >> 16
    g16 = (gu + jnp.uint32(0x8000)) & jnp.uint32(0xFFFF0000)
    words = s16 | g16
    return lax.bitcast_convert_type(words, jnp.int32).reshape(PROWS, H)


def kernel(student_id, problem_id, knowledge_emb, theta_table, slip_table, guess_table):
    fused = _build_fused(slip_table, guess_table)
    theta_g, slip_g, guess_g = _sc_gather(
        theta_table, fused, student_id, problem_id)
    return _tc_math(theta_g, knowledge_emb, slip_g, guess_g)


# speedup vs baseline: 1.1002x; 1.1002x over previous
import dataclasses
import functools

import jax
import jax.numpy as jnp
from jax import lax
from jax.experimental import pallas as pl
from jax.experimental.pallas import tpu as pltpu
from jax.experimental.pallas import tpu_sc as plsc

B = 16384
H = 128
NC = 2
NS = 16
NW = NC * NS
BPW = B // NW
CH = 128
NCH = BPW // CH
L = 16

P = 100000
PROWS = (P + H - 1) // H
FW = 2 * H

MAX_SLIP = 0.4
MAX_GUESS = 0.4
INV_T = 1.0 / 50.0


def _sc_gather(theta_table, fused_table, student_id, problem_id):
    mesh = plsc.VectorSubcoreMesh(core_axis_name="c", subcore_axis_name="s")
    cp = pltpu.CompilerParams()
    if "needs_layout_passes" in pltpu.CompilerParams.__dataclass_fields__:
        cp = dataclasses.replace(cp, needs_layout_passes=False)

    @functools.partial(
        pl.kernel,
        mesh=mesh,
        compiler_params=cp,
        out_type=[
            jax.ShapeDtypeStruct((B, H), jnp.float32),
            jax.ShapeDtypeStruct((B,), jnp.float32),
            jax.ShapeDtypeStruct((B,), jnp.float32),
        ],
        scratch_types=[
            pltpu.VMEM((BPW,), jnp.int32),
            pltpu.VMEM((BPW,), jnp.int32),
            pltpu.VMEM((BPW,), jnp.int32),
            pltpu.VMEM((CH, H), jnp.float32),
            pltpu.VMEM((CH, H), jnp.float32),
            pltpu.VMEM((CH, H), jnp.int32),
            pltpu.VMEM((CH, H), jnp.int32),
            pltpu.VMEM((BPW,), jnp.float32),
            pltpu.VMEM((BPW,), jnp.float32),
            pltpu.SemaphoreType.DMA,
            pltpu.SemaphoreType.DMA,
            pltpu.SemaphoreType.DMA,
            pltpu.SemaphoreType.DMA,
        ],
    )
    def k(theta_hbm, fused_hbm, sid_hbm, pid_hbm,
          theta_out, slip_out, guess_out,
          sidx_v, pidx_v, ridx_v, tbuf0, tbuf1, fbuf0, fbuf1,
          slip_v, guess_v, tsem0, tsem1, fsem0, fsem1):
        wid = lax.axis_index("s") * NC + lax.axis_index("c")
        base = wid * BPW
        pltpu.sync_copy(sid_hbm.at[pl.ds(base, BPW)], sidx_v)
        pltpu.sync_copy(pid_hbm.at[pl.ds(base, BPW)], pidx_v)

        for g in range(BPW // L):
            pv = pidx_v[pl.ds(g * L, L)]
            ridx_v[pl.ds(g * L, L)] = lax.shift_right_logical(pv, 7)

        tbufs = (tbuf0, tbuf1)
        fbufs = (fbuf0, fbuf1)
        tsems = (tsem0, tsem1)
        fsems = (fsem0, fsem1)
        ht = [None] * NCH
        hf = [None] * NCH
        ht[0] = pltpu.async_copy(
            theta_hbm.at[sidx_v.at[pl.ds(0, CH)]], tbufs[0], tsems[0])
        hf[0] = pltpu.async_copy(
            fused_hbm.at[ridx_v.at[pl.ds(0, CH)]], fbufs[0], fsems[0])
        for c in range(NCH):
            if c + 1 < NCH:
                ht[c + 1] = pltpu.async_copy(
                    theta_hbm.at[sidx_v.at[pl.ds((c + 1) * CH, CH)]],
                    tbufs[(c + 1) % 2], tsems[(c + 1) % 2])
                hf[c + 1] = pltpu.async_copy(
                    fused_hbm.at[ridx_v.at[pl.ds((c + 1) * CH, CH)]],
                    fbufs[(c + 1) % 2], fsems[(c + 1) % 2])
            ht[c].wait()
            pltpu.sync_copy(tbufs[c % 2],
                            theta_out.at[pl.ds(base + c * CH, CH)])
            hf[c].wait()
            fb = fbufs[c % 2]
            for g in range(CH // L):
                pv = pidx_v[pl.ds(c * CH + g * L, L)]
                col = jnp.bitwise_and(pv, H - 1)
                rows = lax.iota(jnp.int32, L) + (g * L)
                v = plsc.load_gather(fb, [rows, col])
                slip_v[pl.ds(c * CH + g * L, L)] = plsc.bitcast(
                    lax.shift_left(v, 16), jnp.float32)
                guess_v[pl.ds(c * CH + g * L, L)] = plsc.bitcast(
                    jnp.bitwise_and(v, jnp.int32(-65536)), jnp.float32)
        pltpu.sync_copy(slip_v, slip_out.at[pl.ds(base, BPW)])
        pltpu.sync_copy(guess_v, guess_out.at[pl.ds(base, BPW)])

    return k(theta_table, fused_table, student_id, problem_id)


_TC_BLK = 4096


def _tc_body(t_ref, k_ref, s_ref, g_ref, o_ref):
    th = t_ref[...]
    ke = k_ref[...]
    prod = ke * (jax.nn.sigmoid(th) - 0.5)
    ones = jnp.ones((1, H), jnp.float32)
    n = lax.dot_general(ones, prod, (((1,), (1,)), ((), ())),
                        preferred_element_type=jnp.float32)
    p = jax.nn.sigmoid(n * INV_T)
    s = jax.nn.sigmoid(s_ref[...].reshape(1, _TC_BLK) * MAX_SLIP)
    g = jax.nn.sigmoid(g_ref[...].reshape(1, _TC_BLK) * MAX_GUESS)
    out = (1.0 - s) * p + g * (1.0 - p)
    o_ref[...] = out.reshape(_TC_BLK)


def _tc_math(theta_g, knowledge_emb, slip_g, guess_g):
    return pl.pallas_call(
        _tc_body,
        grid=(B // _TC_BLK,),
        in_specs=[
            pl.BlockSpec((_TC_BLK, H), lambda i: (i, 0)),
            pl.BlockSpec((_TC_BLK, H), lambda i: (i, 0)),
            pl.BlockSpec((_TC_BLK,), lambda i: (i,)),
            pl.BlockSpec((_TC_BLK,), lambda i: (i,)),
        ],
        out_specs=pl.BlockSpec((_TC_BLK,), lambda i: (i,)),
        out_shape=jax.ShapeDtypeStruct((B,), jnp.float32),
    )(theta_g, knowledge_emb, slip_g, guess_g)


def _build_fused(slip_table, guess_table):
    npad = PROWS * H - P
    su = lax.bitcast_convert_type(
        jnp.pad(slip_table[:, 0], (0, npad)), jnp.uint32)
    gu = lax.bitcast_convert_type(
        jnp.pad(guess_table[:, 0], (0, npad)), jnp.uint32)
    s16 = (su + jnp.uint32(0x8000)) >> 16
    g16 = (gu + jnp.uint32(0x8000)) & jnp.uint32(0xFFFF0000)
    words = s16 | g16
    return lax.bitcast_convert_type(words, jnp.int32).reshape(PROWS, H)


def kernel(student_id, problem_id, knowledge_emb, theta_table, slip_table, guess_table):
    fused = _build_fused(slip_table, guess_table)
    theta_g, slip_g, guess_g = _sc_gather(
        theta_table, fused, student_id, problem_id)
    return _tc_math(theta_g, knowledge_emb, slip_g, guess_g)
